# baseline (speedup 1.0000x reference)
import math

import jax
import jax.numpy as jnp
from jax.experimental import pallas as pl
from jax.experimental.pallas import tpu as pltpu
from jax.experimental.pallas import tpu_sc as plsc

OUT_DIM = 128
SCALE = math.sqrt(float(OUT_DIM))
WINDOW = 256
LANES = 16


def kernel(x, table):
    B, L = x.shape
    vocab, out_dim = table.shape
    assert out_dim == OUT_DIM
    n_idx = B * L
    idx_flat = x.reshape(1, n_idx).astype(jnp.int32)

    mesh = plsc.VectorSubcoreMesh(core_axis_name="core",
                                  subcore_axis_name="subcore")

    @pl.kernel(
        out_type=jax.ShapeDtypeStruct((n_idx, OUT_DIM), jnp.float32),
        mesh=mesh,
        compiler_params=pltpu.CompilerParams(use_tc_tiling_on_sc=True),
    )
    def gather_scale(table_hbm, idx_hbm, out_hbm):
        def body(idx_vmem, out_vmem):
            pltpu.sync_copy(table_hbm.at[idx_vmem.at[0]], out_vmem)

            @plsc.parallel_loop(0, WINDOW, unroll=4)
            def _(r):
                for c in range(0, OUT_DIM, LANES):
                    slc = (pl.ds(r, 1), pl.ds(c, LANES))
                    out_vmem.at[*slc][...] = out_vmem.at[*slc][...] * SCALE

        pltpu.emit_pipeline(
            body,
            grid=(n_idx // WINDOW,),
            in_specs=[pl.BlockSpec((1, WINDOW), index_map=lambda i: (0, i))],
            out_specs=[pl.BlockSpec((WINDOW, OUT_DIM),
                                    index_map=lambda i: (i, 0))],
            core_axis_name=("core", "subcore"),
            dimension_semantics=(pltpu.PARALLEL,),
        )(idx_hbm, out_hbm)

    out = gather_scale(table, idx_flat)
    return out.reshape(B, L, OUT_DIM)


# speedup vs baseline: 1.9095x; 1.9095x over previous
import math

import jax
import jax.numpy as jnp
from jax import lax
from jax.experimental import pallas as pl
from jax.experimental.pallas import tpu as pltpu
from jax.experimental.pallas import tpu_sc as plsc

OUT_DIM = 128
SCALE = math.sqrt(float(OUT_DIM))
LANES = 16
NW = 32
CB = 8


def kernel(x, table):
    B, L = x.shape
    vocab, out_dim = table.shape
    assert out_dim == OUT_DIM
    n_idx = B * L
    idx_flat = x.reshape(n_idx).astype(jnp.int32)

    b_per_w = B // NW
    rows = CB * L
    n_chunks = b_per_w // CB

    mesh = plsc.VectorSubcoreMesh(core_axis_name="core",
                                  subcore_axis_name="subcore")

    @pl.kernel(
        out_type=jax.ShapeDtypeStruct((B, L, OUT_DIM), jnp.float32),
        mesh=mesh,
        compiler_params=pltpu.CompilerParams(use_tc_tiling_on_sc=True),
        scratch_types=[
            pltpu.VMEM((b_per_w * L,), jnp.int32),
            pltpu.VMEM((2, rows, OUT_DIM), jnp.float32),
            pltpu.SemaphoreType.DMA,
            pltpu.SemaphoreType.DMA,
            pltpu.SemaphoreType.DMA,
            pltpu.SemaphoreType.DMA,
        ],
    )
    def gather_scale(table_hbm, idx_hbm, out_hbm, idx_v, rows_v,
                     g0, g1, w0, w1):
        wid = lax.axis_index("subcore") * 2 + lax.axis_index("core")
        b0 = wid * b_per_w
        pltpu.sync_copy(idx_hbm.at[pl.ds(b0 * L, b_per_w * L)], idx_v)

        gsems = (g0, g1)
        wsems = (w0, w1)

        def start_gather(g):
            cur = g % 2
            return pltpu.async_copy(
                table_hbm.at[idx_v.at[pl.ds(g * rows, rows)]],
                rows_v.at[cur], gsems[cur])

        def scale_buf(cur):
            @plsc.parallel_loop(0, rows, unroll=4)
            def _(r):
                for c in range(0, OUT_DIM, LANES):
                    slc = (cur, pl.ds(r, 1), pl.ds(c, LANES))
                    rows_v.at[*slc][...] = rows_v.at[*slc][...] * SCALE

        def start_writes(g):
            cur = g % 2
            return [
                pltpu.async_copy(rows_v.at[cur, pl.ds(i * L, L)],
                                 out_hbm.at[b0 + g * CB + i], wsems[cur])
                for i in range(CB)
            ]

        gh = {0: start_gather(0)}
        wh = {}
        for g in range(n_chunks):
            gh[g].wait()
            if g + 1 < n_chunks:
                if g - 1 >= 0:
                    for h in wh[g - 1]:
                        h.wait()
                gh[g + 1] = start_gather(g + 1)
            scale_buf(g % 2)
            wh[g] = start_writes(g)
        for h in wh[n_chunks - 2] + wh[n_chunks - 1]:
            h.wait()

    return gather_scale(table, idx_flat)
